# baseline (speedup 1.0000x reference)
import jax
import jax.numpy as jnp
from jax.experimental import pallas as pl
from jax.experimental.pallas import tpu as pltpu

N = 207
NP = 256
B = 64
U = 64
D0 = 2
D0P = 8
NM = 5
G = 32


def _kernel_body(x0_ref, adj_ref, adjt_ref, w0_ref, b0_ref, w1_ref, b1_ref,
                 h0_ref, h1_ref, s1_ref, s2_ref):
    bf16 = jnp.bfloat16

    @pl.when(pl.program_id(0) == 0)
    def _build_supports():
        adj = adj_ref[...]
        d1 = jnp.sum(adj, axis=1, keepdims=True)
        s1_ref[...] = jnp.where(d1 > 0.0, 1.0 / d1, 0.0) * adj
        adjt = adjt_ref[...]
        d2 = jnp.sum(adjt, axis=1, keepdims=True)
        s2_ref[...] = jnp.where(d2 > 0.0, 1.0 / d2, 0.0) * adjt

    s1 = s1_ref[...]
    s2 = s2_ref[...]

    def dcgru_layer(x0, w, b, din):
        x1a = jnp.dot(s1, x0, preferred_element_type=jnp.float32)
        x2a = 2.0 * jnp.dot(s1, x1a, preferred_element_type=jnp.float32) - x0
        x1b = jnp.dot(s2, x0, preferred_element_type=jnp.float32)
        x2b = 2.0 * jnp.dot(s2, x1b, preferred_element_type=jnp.float32) - x0
        cols = [jnp.concatenate([x[:, g * din:(g + 1) * din]
                                 for g in range(G)], axis=0)
                for x in (x0.astype(bf16), x1a.astype(bf16),
                          x2a.astype(bf16), x1b.astype(bf16),
                          x2b.astype(bf16))]
        z = jnp.concatenate(cols, axis=1)
        gg = jnp.dot(z, w, preferred_element_type=jnp.float32) + b
        tu = jnp.tanh(gg[:, :U])
        c = jnp.tanh(gg[:, U:])
        return (0.5 - 0.5 * tu) * c

    h0t = dcgru_layer(x0_ref[0], w0_ref[...], b0_ref[...], D0P)
    x1in = jnp.concatenate([h0t[g * NP:(g + 1) * NP, :] for g in range(G)],
                           axis=1)
    h1t = dcgru_layer(x1in, w1_ref[...], b1_ref[...], U)
    for g in range(G):
        h0_ref[g] = h0t[g * NP:(g + 1) * NP, :]
        h1_ref[g] = h1t[g * NP:(g + 1) * NP, :]


def _prep_w(Wg, Wc, din, dpad):
    total_in = Wg.shape[0] // NM
    wu = Wg.reshape(total_in, NM, 2 * U)[:din, :, U:]
    wc = Wc.reshape(total_in, NM, U)[:din]
    w = jnp.transpose(jnp.concatenate([0.5 * wu, wc], axis=2), (1, 0, 2))
    if dpad != din:
        w = jnp.pad(w, ((0, 0), (0, dpad - din), (0, 0)))
    return w.reshape(NM * dpad, 2 * U)


def kernel(inputs, adj_mx, nodevec1, nodevec2,
           W_gate_0, b_gate_0, W_cand_0, b_cand_0,
           W_gate_1, b_gate_1, W_cand_1, b_cand_1):
    f32 = jnp.float32
    adj_p = jnp.zeros((NP, NP), f32).at[:N, :N].set(adj_mx)
    adjt_p = adj_p.T
    x0 = jnp.pad(inputs.reshape(B, N, D0),
                 ((0, 0), (0, NP - N), (0, D0P - D0)))
    x0 = jnp.transpose(x0.reshape(B // G, G, NP, D0P),
                       (0, 2, 1, 3)).reshape(B // G, NP, G * D0P)
    w0 = _prep_w(W_gate_0, W_cand_0, D0, D0P).astype(jnp.bfloat16)
    w1 = _prep_w(W_gate_1, W_cand_1, U, U).astype(jnp.bfloat16)
    b0 = jnp.concatenate([0.5 * b_gate_0[U:], b_cand_0]).reshape(1, 2 * U)
    b1 = jnp.concatenate([0.5 * b_gate_1[U:], b_cand_1]).reshape(1, 2 * U)

    full = lambda shape: pl.BlockSpec(shape, lambda c: (0,) * len(shape))
    h0, h1 = pl.pallas_call(
        _kernel_body,
        grid=(B // G,),
        in_specs=[
            pl.BlockSpec((1, NP, G * D0P), lambda c: (c, 0, 0)),
            full((NP, NP)), full((NP, NP)),
            full((NM * D0P, 2 * U)), full((1, 2 * U)),
            full((NM * U, 2 * U)), full((1, 2 * U)),
        ],
        out_specs=[pl.BlockSpec((G, NP, U), lambda c: (c, 0, 0))] * 2,
        out_shape=[jax.ShapeDtypeStruct((B, NP, U), f32)] * 2,
        scratch_shapes=[pltpu.VMEM((NP, NP), f32)] * 2,
    )(x0, adj_p, adjt_p, w0, b0, w1, b1)

    h0f = h0[:, :N, :].reshape(B, N * U)
    h1f = h1[:, :N, :].reshape(B, N * U)
    return h1f, jnp.stack([h0f, h1f])


# speedup vs baseline: 1.0787x; 1.0787x over previous
import jax
import jax.numpy as jnp
from jax.experimental import pallas as pl
from jax.experimental.pallas import tpu as pltpu

N = 207
NP = 256
B = 64
U = 64
D0 = 2
D0P = 8
NM = 5
G = 32


def _kernel_body(x0_ref, adj_ref, adjt_ref, w0_ref, b0_ref, w1_ref, b1_ref,
                 h0_ref, h1_ref, s1_ref, s2_ref):
    bf16 = jnp.bfloat16

    @pl.when(pl.program_id(0) == 0)
    def _build_supports():
        adj = adj_ref[...]
        d1 = jnp.sum(adj, axis=1, keepdims=True)
        s1_ref[...] = jnp.where(d1 > 0.0, 1.0 / d1, 0.0) * adj
        adjt = adjt_ref[...]
        d2 = jnp.sum(adjt, axis=1, keepdims=True)
        s2_ref[...] = jnp.where(d2 > 0.0, 1.0 / d2, 0.0) * adjt

    s1 = s1_ref[...]
    s2 = s2_ref[...]

    def dcgru_layer(x0, w, b, din):
        P = 128 // din
        C = G // P
        x1a = jnp.dot(s1, x0, preferred_element_type=jnp.float32)
        x2a = 2.0 * jnp.dot(s1, x1a, preferred_element_type=jnp.float32) - x0
        x1b = jnp.dot(s2, x0, preferred_element_type=jnp.float32)
        x2b = 2.0 * jnp.dot(s2, x1b, preferred_element_type=jnp.float32) - x0
        cols = [jnp.concatenate([x[:, c * 128:(c + 1) * 128]
                                 for c in range(C)], axis=0).astype(bf16)
                for x in (x0, x1a, x2a, x1b, x2b)]
        z = jnp.concatenate(cols, axis=1)
        gg = jnp.dot(z, w, preferred_element_type=jnp.float32) + b
        th = jnp.tanh(gg)
        ths = jnp.concatenate([th[:, 64:], th[:, :64]], axis=1)
        hv = (0.5 - 0.5 * th) * ths
        return [hv[c * NP:(c + 1) * NP, e * 128:e * 128 + 64]
                for c in range(C) for e in range(P)]

    h0s = dcgru_layer(x0_ref[0], w0_ref[...], b0_ref[...], D0P)
    x1in = jnp.concatenate(h0s, axis=1)
    h1s = dcgru_layer(x1in, w1_ref[...], b1_ref[...], U)
    for g in range(G):
        h0_ref[g] = h0s[g]
        h1_ref[g] = h1s[g]


def _prep_w(Wg, Wc, din, dpad):
    total_in = Wg.shape[0] // NM
    wu = Wg.reshape(total_in, NM, 2 * U)[:din, :, U:]
    wc = Wc.reshape(total_in, NM, U)[:din]
    w = jnp.transpose(jnp.concatenate([0.5 * wu, wc], axis=2), (1, 0, 2))
    if dpad != din:
        w = jnp.pad(w, ((0, 0), (0, dpad - din), (0, 0)))
    P = 128 // dpad
    wbd = jnp.zeros((NM, P, dpad, P, 2 * U), w.dtype)
    for e in range(P):
        wbd = wbd.at[:, e, :, e, :].set(w)
    return wbd.reshape(NM * 128, P * 128).astype(jnp.bfloat16)


def kernel(inputs, adj_mx, nodevec1, nodevec2,
           W_gate_0, b_gate_0, W_cand_0, b_cand_0,
           W_gate_1, b_gate_1, W_cand_1, b_cand_1):
    f32 = jnp.float32
    adj_p = jnp.zeros((NP, NP), f32).at[:N, :N].set(adj_mx)
    adjt_p = adj_p.T
    x0 = jnp.pad(inputs.reshape(B, N, D0),
                 ((0, 0), (0, NP - N), (0, D0P - D0)))
    x0 = jnp.transpose(x0.reshape(B // G, G, NP, D0P),
                       (0, 2, 1, 3)).reshape(B // G, NP, G * D0P)
    w0 = _prep_w(W_gate_0, W_cand_0, D0, D0P)
    w1 = _prep_w(W_gate_1, W_cand_1, U, U)
    bf0 = jnp.concatenate([0.5 * b_gate_0[U:], b_cand_0])
    bf1 = jnp.concatenate([0.5 * b_gate_1[U:], b_cand_1])
    b0 = jnp.tile(bf0, 128 // D0P).reshape(1, (128 // D0P) * 2 * U)
    b1 = jnp.tile(bf1, 128 // U).reshape(1, (128 // U) * 2 * U)

    full = lambda shape: pl.BlockSpec(shape, lambda c: (0,) * len(shape))
    h0, h1 = pl.pallas_call(
        _kernel_body,
        grid=(B // G,),
        in_specs=[
            pl.BlockSpec((1, NP, G * D0P), lambda c: (c, 0, 0)),
            full((NP, NP)), full((NP, NP)),
            full((NM * 128, (128 // D0P) * 2 * U)),
            full((1, (128 // D0P) * 2 * U)),
            full((NM * 128, (128 // U) * 2 * U)),
            full((1, (128 // U) * 2 * U)),
        ],
        out_specs=[pl.BlockSpec((G, NP, U), lambda c: (c, 0, 0))] * 2,
        out_shape=[jax.ShapeDtypeStruct((B, NP, U), f32)] * 2,
        scratch_shapes=[pltpu.VMEM((NP, NP), f32)] * 2,
    )(x0, adj_p, adjt_p, w0, b0, w1, b1)

    h0f = h0[:, :N, :].reshape(B, N * U)
    h1f = h1[:, :N, :].reshape(B, N * U)
    return h1f, jnp.stack([h0f, h1f])


# speedup vs baseline: 1.0819x; 1.0030x over previous
import jax
import jax.numpy as jnp
from jax.experimental import pallas as pl
from jax.experimental.pallas import tpu as pltpu

N = 207
NP = 256
B = 64
U = 64
D0 = 2
D0P = 8
NM = 5
G = 32


def _kernel_body(x0_ref, adj_ref, adjt_ref, w0_ref, b0_ref, w1_ref, b1_ref,
                 h0_ref, h1_ref):
    bf16 = jnp.bfloat16

    adj = adj_ref[...]
    d1 = jnp.sum(adj, axis=1, keepdims=True)
    s1 = jnp.where(d1 > 0.0, 1.0 / d1, 0.0) * adj
    adjt = adjt_ref[...]
    d2 = jnp.sum(adjt, axis=1, keepdims=True)
    s2 = jnp.where(d2 > 0.0, 1.0 / d2, 0.0) * adjt

    def dcgru_layer(x0, w, b, din):
        P = 128 // din
        C = G // P
        x1a = jnp.dot(s1, x0, preferred_element_type=jnp.float32)
        x2a = 2.0 * jnp.dot(s1, x1a, preferred_element_type=jnp.float32) - x0
        x1b = jnp.dot(s2, x0, preferred_element_type=jnp.float32)
        x2b = 2.0 * jnp.dot(s2, x1b, preferred_element_type=jnp.float32) - x0
        cols = [jnp.concatenate([x[:, c * 128:(c + 1) * 128]
                                 for c in range(C)], axis=0).astype(bf16)
                for x in (x0, x1a, x2a, x1b, x2b)]
        z = jnp.concatenate(cols, axis=1)
        gg = jnp.dot(z, w, preferred_element_type=jnp.float32) + b
        th = jnp.tanh(gg)
        ths = jnp.concatenate([th[:, 64:], th[:, :64]], axis=1)
        hv = (0.5 - 0.5 * th) * ths
        return [hv[c * NP:(c + 1) * NP, e * 128:e * 128 + 64]
                for c in range(C) for e in range(P)]

    h0s = dcgru_layer(x0_ref[0], w0_ref[...], b0_ref[...], D0P)
    x1in = jnp.concatenate(h0s, axis=1)
    h1s = dcgru_layer(x1in, w1_ref[...], b1_ref[...], U)
    for g in range(G):
        h0_ref[g] = h0s[g]
        h1_ref[g] = h1s[g]


def _prep_w(Wg, Wc, din, dpad):
    total_in = Wg.shape[0] // NM
    wu = Wg.reshape(total_in, NM, 2 * U)[:din, :, U:]
    wc = Wc.reshape(total_in, NM, U)[:din]
    w = jnp.transpose(jnp.concatenate([0.5 * wu, wc], axis=2), (1, 0, 2))
    if dpad != din:
        w = jnp.pad(w, ((0, 0), (0, dpad - din), (0, 0)))
    P = 128 // dpad
    wbd = jnp.zeros((NM, P, dpad, P, 2 * U), w.dtype)
    for e in range(P):
        wbd = wbd.at[:, e, :, e, :].set(w)
    return wbd.reshape(NM * 128, P * 128).astype(jnp.bfloat16)


def kernel(inputs, adj_mx, nodevec1, nodevec2,
           W_gate_0, b_gate_0, W_cand_0, b_cand_0,
           W_gate_1, b_gate_1, W_cand_1, b_cand_1):
    f32 = jnp.float32
    adj_p = jnp.zeros((NP, NP), f32).at[:N, :N].set(adj_mx)
    adjt_p = adj_p.T
    x0 = jnp.pad(inputs.reshape(B, N, D0),
                 ((0, 0), (0, NP - N), (0, D0P - D0)))
    x0 = jnp.transpose(x0.reshape(B // G, G, NP, D0P),
                       (0, 2, 1, 3)).reshape(B // G, NP, G * D0P)
    w0 = _prep_w(W_gate_0, W_cand_0, D0, D0P)
    w1 = _prep_w(W_gate_1, W_cand_1, U, U)
    bf0 = jnp.concatenate([0.5 * b_gate_0[U:], b_cand_0])
    bf1 = jnp.concatenate([0.5 * b_gate_1[U:], b_cand_1])
    b0 = jnp.tile(bf0, 128 // D0P).reshape(1, (128 // D0P) * 2 * U)
    b1 = jnp.tile(bf1, 128 // U).reshape(1, (128 // U) * 2 * U)

    full = lambda shape: pl.BlockSpec(shape, lambda c: (0,) * len(shape))
    h0, h1 = pl.pallas_call(
        _kernel_body,
        grid=(B // G,),
        in_specs=[
            pl.BlockSpec((1, NP, G * D0P), lambda c: (c, 0, 0)),
            full((NP, NP)), full((NP, NP)),
            full((NM * 128, (128 // D0P) * 2 * U)),
            full((1, (128 // D0P) * 2 * U)),
            full((NM * 128, (128 // U) * 2 * U)),
            full((1, (128 // U) * 2 * U)),
        ],
        out_specs=[pl.BlockSpec((G, NP, U), lambda c: (c, 0, 0))] * 2,
        out_shape=[jax.ShapeDtypeStruct((B, NP, U), f32)] * 2,
        compiler_params=pltpu.CompilerParams(
            dimension_semantics=("parallel",)),
    )(x0, adj_p, adjt_p, w0, b0, w1, b1)

    h0f = h0[:, :N, :].reshape(B, N * U)
    h1f = h1[:, :N, :].reshape(B, N * U)
    return h1f, jnp.stack([h0f, h1f])


# speedup vs baseline: 1.2317x; 1.1385x over previous
import jax
import jax.numpy as jnp
from jax.experimental import pallas as pl
from jax.experimental.pallas import tpu as pltpu

N = 207
NP = 256
B = 64
U = 64
D0 = 2
D0P = 8
NM = 5
G = 32


def _kernel_body(x0_ref, adj_ref, w0_ref, b0_ref, w1_ref, b1_ref, h_ref):
    bf16 = jnp.bfloat16

    adj = adj_ref[...]
    d1 = jnp.sum(adj, axis=1, keepdims=True)
    s1 = jnp.where(d1 > 0.0, 1.0 / d1, 0.0) * adj
    adjt = jnp.transpose(adj)
    d2 = jnp.sum(adjt, axis=1, keepdims=True)
    s2 = jnp.where(d2 > 0.0, 1.0 / d2, 0.0) * adjt

    def dcgru_layer(x0, w, b, din):
        P = 128 // din
        C = G // P
        x1a = jnp.dot(s1, x0, preferred_element_type=jnp.float32)
        x2a = 2.0 * jnp.dot(s1, x1a, preferred_element_type=jnp.float32) - x0
        x1b = jnp.dot(s2, x0, preferred_element_type=jnp.float32)
        x2b = 2.0 * jnp.dot(s2, x1b, preferred_element_type=jnp.float32) - x0
        cols = [jnp.concatenate([x[:, c * 128:(c + 1) * 128]
                                 for c in range(C)], axis=0).astype(bf16)
                for x in (x0, x1a, x2a, x1b, x2b)]
        z = jnp.concatenate(cols, axis=1)
        gg = jnp.dot(z, w, preferred_element_type=jnp.float32) + b
        th = jnp.tanh(gg)
        ths = jnp.concatenate([th[:, 64:], th[:, :64]], axis=1)
        hv = (0.5 - 0.5 * th) * ths
        return [hv[c * NP:(c + 1) * NP, e * 128:e * 128 + 64]
                for c in range(C) for e in range(P)]

    h0s = dcgru_layer(x0_ref[0], w0_ref[...], b0_ref[...], D0P)
    x1in = jnp.concatenate(h0s, axis=1)
    h1s = dcgru_layer(x1in, w1_ref[...], b1_ref[...], U)
    for g in range(G):
        h_ref[0, g] = h0s[g]
        h_ref[1, g] = h1s[g]


def _prep_w(Wg, Wc, din, dpad):
    total_in = Wg.shape[0] // NM
    wu = Wg.reshape(total_in, NM, 2 * U)[:din, :, U:]
    wc = Wc.reshape(total_in, NM, U)[:din]
    w = jnp.transpose(jnp.concatenate([0.5 * wu, wc], axis=2), (1, 0, 2))
    if dpad != din:
        w = jnp.pad(w, ((0, 0), (0, dpad - din), (0, 0)))
    P = 128 // dpad
    wbd = (w[:, None, :, None, :]
           * jnp.eye(P, dtype=w.dtype)[None, :, None, :, None])
    return wbd.reshape(NM * 128, P * 128).astype(jnp.bfloat16)


def kernel(inputs, adj_mx, nodevec1, nodevec2,
           W_gate_0, b_gate_0, W_cand_0, b_cand_0,
           W_gate_1, b_gate_1, W_cand_1, b_cand_1):
    f32 = jnp.float32
    adj_p = jnp.zeros((NP, NP), f32).at[:N, :N].set(adj_mx)
    x0 = jnp.pad(inputs.reshape(B, N, D0),
                 ((0, 0), (0, NP - N), (0, D0P - D0)))
    x0 = jnp.transpose(x0.reshape(B // G, G, NP, D0P),
                       (0, 2, 1, 3)).reshape(B // G, NP, G * D0P)
    w0 = _prep_w(W_gate_0, W_cand_0, D0, D0P)
    w1 = _prep_w(W_gate_1, W_cand_1, U, U)
    bf0 = jnp.concatenate([0.5 * b_gate_0[U:], b_cand_0])
    bf1 = jnp.concatenate([0.5 * b_gate_1[U:], b_cand_1])
    b0 = jnp.tile(bf0, 128 // D0P).reshape(1, (128 // D0P) * 2 * U)
    b1 = jnp.tile(bf1, 128 // U).reshape(1, (128 // U) * 2 * U)

    full = lambda shape: pl.BlockSpec(shape, lambda c: (0,) * len(shape))
    h = pl.pallas_call(
        _kernel_body,
        grid=(B // G,),
        in_specs=[
            pl.BlockSpec((1, NP, G * D0P), lambda c: (c, 0, 0)),
            full((NP, NP)),
            full((NM * 128, (128 // D0P) * 2 * U)),
            full((1, (128 // D0P) * 2 * U)),
            full((NM * 128, (128 // U) * 2 * U)),
            full((1, (128 // U) * 2 * U)),
        ],
        out_specs=pl.BlockSpec((2, G, NP, U), lambda c: (0, c, 0, 0)),
        out_shape=jax.ShapeDtypeStruct((2, B, NP, U), f32),
        compiler_params=pltpu.CompilerParams(
            dimension_semantics=("parallel",)),
    )(x0, adj_p, w0, b0, w1, b1)

    hidden = h[:, :, :N, :].reshape(2, B, N * U)
    return hidden[1], hidden


# speedup vs baseline: 1.4354x; 1.1653x over previous
import jax
import jax.numpy as jnp
from jax.experimental import pallas as pl
from jax.experimental.pallas import tpu as pltpu

N = 207
NP = 256
B = 64
U = 64
D0 = 2
D0P = 8
NM = 5
G = 32


def _kernel_body(x0_ref, adj_ref, wg0_ref, wc0_ref, bg0_ref, bc0_ref,
                 wg1_ref, wc1_ref, bg1_ref, bc1_ref, h_ref,
                 s1_ref, s2_ref, w0_ref, b0_ref, w1_ref, b1_ref):
    bf16 = jnp.bfloat16

    @pl.when(pl.program_id(0) == 0)
    def _prep():
        adj = adj_ref[...]
        d1 = jnp.sum(adj, axis=1, keepdims=True)
        s1_ref[...] = jnp.where(d1 > 0.0, 1.0 / d1, 0.0) * adj
        adjt = jnp.transpose(adj)
        d2 = jnp.sum(adjt, axis=1, keepdims=True)
        s2_ref[...] = jnp.where(d2 > 0.0, 1.0 / d2, 0.0) * adjt

        def prep_w(wg_ref, wc_ref, din, dpad, wbd_ref):
            total_in = din + U
            wu = wg_ref[...].reshape(total_in, NM, 2 * U)[:din, :, U:]
            wc = wc_ref[...].reshape(total_in, NM, U)[:din]
            w = jnp.concatenate([0.5 * wu, wc], axis=2).astype(bf16)
            P = 128 // dpad
            wbd_ref[...] = jnp.zeros_like(wbd_ref)
            for m in range(NM):
                for e in range(P):
                    r = m * 128 + e * dpad
                    wbd_ref[r:r + din, e * 128:(e + 1) * 128] = w[:, m, :]

        prep_w(wg0_ref, wc0_ref, D0, D0P, w0_ref)
        prep_w(wg1_ref, wc1_ref, U, U, w1_ref)
        bf0 = jnp.concatenate([0.5 * bg0_ref[:, U:], bc0_ref[...]], axis=1)
        bf1 = jnp.concatenate([0.5 * bg1_ref[:, U:], bc1_ref[...]], axis=1)
        b0_ref[...] = jnp.concatenate([bf0] * (128 // D0P), axis=1)
        b1_ref[...] = jnp.concatenate([bf1] * (128 // U), axis=1)

    s1 = s1_ref[...]
    s2 = s2_ref[...]

    def dcgru_layer(x0, w, b, din):
        P = 128 // din
        C = G // P
        x1a = jnp.dot(s1, x0, preferred_element_type=jnp.float32)
        x2a = 2.0 * jnp.dot(s1, x1a, preferred_element_type=jnp.float32) - x0
        x1b = jnp.dot(s2, x0, preferred_element_type=jnp.float32)
        x2b = 2.0 * jnp.dot(s2, x1b, preferred_element_type=jnp.float32) - x0
        cols = [jnp.concatenate([x[:, c * 128:(c + 1) * 128]
                                 for c in range(C)], axis=0).astype(bf16)
                for x in (x0, x1a, x2a, x1b, x2b)]
        z = jnp.concatenate(cols, axis=1)
        gg = jnp.dot(z, w, preferred_element_type=jnp.float32) + b
        th = jnp.tanh(gg)
        ths = jnp.concatenate([th[:, 64:], th[:, :64]], axis=1)
        hv = (0.5 - 0.5 * th) * ths
        return [hv[c * NP:(c + 1) * NP, e * 128:e * 128 + 64]
                for c in range(C) for e in range(P)]

    h0s = dcgru_layer(x0_ref[0], w0_ref[...], b0_ref[...], D0P)
    x1in = jnp.concatenate(h0s, axis=1)
    h1s = dcgru_layer(x1in, w1_ref[...], b1_ref[...], U)
    for g in range(G):
        h_ref[0, g] = h0s[g]
        h_ref[1, g] = h1s[g]


def kernel(inputs, adj_mx, nodevec1, nodevec2,
           W_gate_0, b_gate_0, W_cand_0, b_cand_0,
           W_gate_1, b_gate_1, W_cand_1, b_cand_1):
    f32 = jnp.float32
    bf16 = jnp.bfloat16
    adj_p = jnp.zeros((NP, NP), f32).at[:N, :N].set(adj_mx)
    x0 = jnp.pad(inputs.reshape(B, N, D0),
                 ((0, 0), (0, NP - N), (0, D0P - D0)))
    x0 = jnp.transpose(x0.reshape(B // G, G, NP, D0P),
                       (0, 2, 1, 3)).reshape(B // G, NP, G * D0P)

    full = lambda shape: pl.BlockSpec(shape, lambda c: (0,) * len(shape))
    h = pl.pallas_call(
        _kernel_body,
        grid=(B // G,),
        in_specs=[
            pl.BlockSpec((1, NP, G * D0P), lambda c: (c, 0, 0)),
            full((NP, NP)),
            full(((D0 + U) * NM, 2 * U)), full(((D0 + U) * NM, U)),
            full((1, 2 * U)), full((1, U)),
            full((2 * U * NM, 2 * U)), full((2 * U * NM, U)),
            full((1, 2 * U)), full((1, U)),
        ],
        out_specs=pl.BlockSpec((2, G, NP, U), lambda c: (0, c, 0, 0)),
        out_shape=jax.ShapeDtypeStruct((2, B, NP, U), f32),
        scratch_shapes=[
            pltpu.VMEM((NP, NP), f32), pltpu.VMEM((NP, NP), f32),
            pltpu.VMEM((NM * 128, (128 // D0P) * 2 * U), bf16),
            pltpu.VMEM((1, (128 // D0P) * 2 * U), f32),
            pltpu.VMEM((NM * 128, (128 // U) * 2 * U), bf16),
            pltpu.VMEM((1, (128 // U) * 2 * U), f32),
        ],
    )(x0, adj_p,
      W_gate_0, W_cand_0, b_gate_0.reshape(1, 2 * U), b_cand_0.reshape(1, U),
      W_gate_1, W_cand_1, b_gate_1.reshape(1, 2 * U), b_cand_1.reshape(1, U))

    hidden = h[:, :, :N, :].reshape(2, B, N * U)
    return hidden[1], hidden


# speedup vs baseline: 1.9143x; 1.3337x over previous
import jax
import jax.numpy as jnp
from jax.experimental import pallas as pl
from jax.experimental.pallas import tpu as pltpu

N = 207
NP = 256
B = 64
U = 64
D0 = 2
D0P = 8
NM = 5
G = 32


def _kernel_body(x0_ref, adj_ref, wg0_ref, wc0_ref, bg0_ref, bc0_ref,
                 wg1_ref, wc1_ref, bg1_ref, bc1_ref, h_ref,
                 s1_ref, s2_ref, w0_ref, b0_ref, w1_ref, b1_ref):
    bf16 = jnp.bfloat16

    @pl.when(pl.program_id(0) == 0)
    def _prep():
        valid = ((jax.lax.broadcasted_iota(jnp.int32, (NP, NP), 0) < N)
                 & (jax.lax.broadcasted_iota(jnp.int32, (NP, NP), 1) < N))
        adj = jnp.where(valid, adj_ref[...], 0.0)
        d1 = jnp.sum(adj, axis=1, keepdims=True)
        s1_ref[...] = jnp.where(d1 > 0.0, 1.0 / d1, 0.0) * adj
        adjt = jnp.transpose(adj)
        d2 = jnp.sum(adjt, axis=1, keepdims=True)
        s2_ref[...] = jnp.where(d2 > 0.0, 1.0 / d2, 0.0) * adjt

        def prep_w(wg_ref, wc_ref, din, dpad, wbd_ref):
            total_in = din + U
            wu = wg_ref[...].reshape(total_in, NM, 2 * U)[:din, :, U:]
            wc = wc_ref[...].reshape(total_in, NM, U)[:din]
            w = jnp.concatenate([0.5 * wu, wc], axis=2).astype(bf16)
            P = 128 // dpad
            wbd_ref[...] = jnp.zeros_like(wbd_ref)
            for m in range(NM):
                for e in range(P):
                    r = m * 128 + e * dpad
                    wbd_ref[r:r + din, e * 128:(e + 1) * 128] = w[:, m, :]

        prep_w(wg0_ref, wc0_ref, D0, D0P, w0_ref)
        prep_w(wg1_ref, wc1_ref, U, U, w1_ref)
        bf0 = jnp.concatenate([0.5 * bg0_ref[:, U:], bc0_ref[...]], axis=1)
        bf1 = jnp.concatenate([0.5 * bg1_ref[:, U:], bc1_ref[...]], axis=1)
        b0_ref[...] = jnp.concatenate([bf0] * (128 // D0P), axis=1)
        b1_ref[...] = jnp.concatenate([bf1] * (128 // U), axis=1)

    s1 = s1_ref[...]
    s2 = s2_ref[...]

    def dcgru_layer(x0, w, b, din):
        P = 128 // din
        C = G // P
        x1a = jnp.dot(s1, x0, preferred_element_type=jnp.float32)
        x2a = 2.0 * jnp.dot(s1, x1a, preferred_element_type=jnp.float32) - x0
        x1b = jnp.dot(s2, x0, preferred_element_type=jnp.float32)
        x2b = 2.0 * jnp.dot(s2, x1b, preferred_element_type=jnp.float32) - x0
        cols = [jnp.concatenate([x[:, c * 128:(c + 1) * 128]
                                 for c in range(C)], axis=0).astype(bf16)
                for x in (x0, x1a, x2a, x1b, x2b)]
        z = jnp.concatenate(cols, axis=1)
        gg = jnp.dot(z, w, preferred_element_type=jnp.float32) + b
        th = jnp.tanh(gg)
        ths = jnp.concatenate([th[:, 64:], th[:, :64]], axis=1)
        hv = (0.5 - 0.5 * th) * ths
        return [hv[c * NP:(c + 1) * NP, e * 128:e * 128 + 64]
                for c in range(C) for e in range(P)]

    h0s = dcgru_layer(x0_ref[0], w0_ref[...], b0_ref[...], D0P)
    x1in = jnp.concatenate(h0s, axis=1)
    h1s = dcgru_layer(x1in, w1_ref[...], b1_ref[...], U)
    for g in range(G):
        h_ref[0, g] = h0s[g]
        h_ref[1, g] = h1s[g]


def kernel(inputs, adj_mx, nodevec1, nodevec2,
           W_gate_0, b_gate_0, W_cand_0, b_cand_0,
           W_gate_1, b_gate_1, W_cand_1, b_cand_1):
    f32 = jnp.float32
    bf16 = jnp.bfloat16
    x0 = jnp.pad(inputs.reshape(B, N, D0),
                 ((0, 0), (0, NP - N), (0, D0P - D0)))
    x0 = jnp.transpose(x0.reshape(B // G, G, NP, D0P),
                       (0, 2, 1, 3)).reshape(B // G, NP, G * D0P)

    full = lambda shape: pl.BlockSpec(shape, lambda c: (0,) * len(shape))
    h = pl.pallas_call(
        _kernel_body,
        grid=(B // G,),
        in_specs=[
            pl.BlockSpec((1, NP, G * D0P), lambda c: (c, 0, 0)),
            full((NP, NP)),
            full(((D0 + U) * NM, 2 * U)), full(((D0 + U) * NM, U)),
            full((1, 2 * U)), full((1, U)),
            full((2 * U * NM, 2 * U)), full((2 * U * NM, U)),
            full((1, 2 * U)), full((1, U)),
        ],
        out_specs=pl.BlockSpec((2, G, NP, U), lambda c: (0, c, 0, 0)),
        out_shape=jax.ShapeDtypeStruct((2, B, N, U), f32),
        scratch_shapes=[
            pltpu.VMEM((NP, NP), f32), pltpu.VMEM((NP, NP), f32),
            pltpu.VMEM((NM * 128, (128 // D0P) * 2 * U), bf16),
            pltpu.VMEM((1, (128 // D0P) * 2 * U), f32),
            pltpu.VMEM((NM * 128, (128 // U) * 2 * U), bf16),
            pltpu.VMEM((1, (128 // U) * 2 * U), f32),
        ],
    )(x0, adj_mx,
      W_gate_0, W_cand_0, b_gate_0.reshape(1, 2 * U), b_cand_0.reshape(1, U),
      W_gate_1, W_cand_1, b_gate_1.reshape(1, 2 * U), b_cand_1.reshape(1, U))

    hidden = h.reshape(2, B, N * U)
    return hidden[1], hidden
